# Initial kernel scaffold; baseline (speedup 1.0000x reference)
import functools

import jax
import jax.numpy as jnp
from jax import lax
from jax.experimental import pallas as pl
from jax.experimental.pallas import tpu as pltpu
from jax.experimental.pallas import tpu_sc as plsc

N_USERS = 50000
N_ITEMS = 50000
NTOT = N_USERS + N_ITEMS
EMB = 32
BATCH = 4096
TAU1 = 0.07
TAU2 = 0.1
W_LAMBDA = 0.5
DECAY = 0.0001

NC = 2
NS = 16
LANES = 16

E_PER_DIR = 800000
E_PER_TILE = E_PER_DIR // NS
CHUNK = 80
GROUPS = CHUNK // LANES
N_CHUNKS = E_PER_TILE // CHUNK
HALF = N_USERS
ROWS_PER_TILE = HALF // NS
WB = 625
WB_STEPS = ROWS_PER_TILE // WB

_mesh = plsc.VectorSubcoreMesh(core_axis_name="c", subcore_axis_name="s")


@functools.partial(
    pl.kernel,
    out_type=jax.ShapeDtypeStruct((NTOT, EMB), jnp.float32),
    mesh=_mesh,
    scratch_types=[
        pltpu.VMEM((CHUNK,), jnp.int32),
        pltpu.VMEM((CHUNK,), jnp.int32),
        pltpu.VMEM((CHUNK,), jnp.float32),
        pltpu.VMEM((CHUNK, EMB), jnp.float32),
        pltpu.VMEM((WB, EMB), jnp.float32),
        pltpu.VMEM_SHARED((HALF, EMB), jnp.float32),
        pltpu.SemaphoreType.DMA,
    ],
)
def _layer(src, dst, val, tin, tout, idx_s, idx_d, valv, rows, zwb, acc, sem):
    c = lax.axis_index("c")
    s = lax.axis_index("s")
    zero16 = jnp.zeros((LANES,), jnp.float32)
    iota16 = lax.broadcasted_iota(jnp.int32, (LANES,), 0)

    def _zrow(i, carry):
        zwb[i, pl.ds(0, LANES)] = zero16
        zwb[i, pl.ds(LANES, LANES)] = zero16
        return carry

    lax.fori_loop(0, WB, _zrow, 0)
    row0 = s * ROWS_PER_TILE
    for k in range(WB_STEPS):
        pltpu.sync_copy(zwb, acc.at[pl.ds(row0 + k * WB, WB), :])
    plsc.subcore_barrier()

    dbase = jnp.where(c == 0, N_USERS, 0).astype(jnp.int32)
    ebase = c * E_PER_DIR + s * E_PER_TILE

    def _chunk(i, carry):
        off = ebase + i * CHUNK
        pltpu.sync_copy(src.at[pl.ds(off, CHUNK)], idx_s)
        pltpu.sync_copy(dst.at[pl.ds(off, CHUNK)], idx_d)
        pltpu.sync_copy(val.at[pl.ds(off, CHUNK)], valv)
        pltpu.async_copy(tin.at[idx_s], rows, sem).wait()
        for g in range(GROUPS):
            gs = g * LANES
            idx_d[pl.ds(gs, LANES)] = idx_d[pl.ds(gs, LANES)] - dbase
            v = valv[pl.ds(gs, LANES)]
            rowids = gs + iota16
            for d in range(EMB):
                colids = jnp.full((LANES,), d, jnp.int32)
                x = plsc.load_gather(rows, [rowids, colids])
                plsc.store_scatter(rows, [rowids, colids], x * v)
        pltpu.sync_copy(rows, acc.at[idx_d], add=True)
        return carry

    lax.fori_loop(0, N_CHUNKS, _chunk, 0)
    plsc.subcore_barrier()

    obase = jnp.where(c == 0, N_USERS, 0) + row0
    for k in range(WB_STEPS):
        pltpu.sync_copy(acc.at[pl.ds(row0 + k * WB, WB), :], zwb)
        pltpu.sync_copy(zwb, tout.at[pl.ds(obase + k * WB, WB), :])


B_PER_W = BATCH // (NC * NS)


@functools.partial(
    pl.kernel,
    out_type=(
        jax.ShapeDtypeStruct((BATCH, EMB), jnp.float32),
        jax.ShapeDtypeStruct((BATCH, EMB), jnp.float32),
        jax.ShapeDtypeStruct((BATCH, EMB), jnp.float32),
        jax.ShapeDtypeStruct((BATCH, EMB), jnp.float32),
        jax.ShapeDtypeStruct((BATCH, EMB), jnp.float32),
        jax.ShapeDtypeStruct((BATCH, EMB), jnp.float32),
    ),
    mesh=_mesh,
    scratch_types=[
        pltpu.VMEM((B_PER_W,), jnp.int32),
        pltpu.VMEM((B_PER_W, EMB), jnp.float32),
        pltpu.VMEM((B_PER_W, EMB), jnp.float32),
        pltpu.SemaphoreType.DMA,
    ],
)
def _batch_gather(t0, t1, t2, t3, users, items, upop, ipop, eup, eip,
                  su, si, u0, p0, upr, ppr, idxv, accv, tmpv, sem):
    c = lax.axis_index("c")
    s = lax.axis_index("s")
    wid = c * NS + s
    base = wid * B_PER_W

    def _acc_add(i, carry):
        accv[i, pl.ds(0, LANES)] = accv[i, pl.ds(0, LANES)] + tmpv[i, pl.ds(0, LANES)]
        accv[i, pl.ds(LANES, LANES)] = (
            accv[i, pl.ds(LANES, LANES)] + tmpv[i, pl.ds(LANES, LANES)]
        )
        return carry

    def _sum4(out_sum, out_first):
        pltpu.async_copy(t0.at[idxv], accv, sem).wait()
        if out_first is not None:
            pltpu.sync_copy(accv, out_first.at[pl.ds(base, B_PER_W), :])
        for t in (t1, t2, t3):
            pltpu.async_copy(t.at[idxv], tmpv, sem).wait()
            lax.fori_loop(0, B_PER_W, _acc_add, 0)
        pltpu.sync_copy(accv, out_sum.at[pl.ds(base, B_PER_W), :])

    pltpu.sync_copy(users.at[pl.ds(base, B_PER_W)], idxv)
    _sum4(su, u0)

    pltpu.sync_copy(items.at[pl.ds(base, B_PER_W)], idxv)
    for g in range(B_PER_W // LANES):
        gs = g * LANES
        idxv[pl.ds(gs, LANES)] = idxv[pl.ds(gs, LANES)] + N_USERS
    _sum4(si, p0)

    pltpu.sync_copy(upop.at[pl.ds(base, B_PER_W)], idxv)
    pltpu.async_copy(eup.at[idxv], tmpv, sem).wait()
    pltpu.sync_copy(tmpv, upr.at[pl.ds(base, B_PER_W), :])
    pltpu.sync_copy(ipop.at[pl.ds(base, B_PER_W)], idxv)
    pltpu.async_copy(eip.at[idxv], tmpv, sem).wait()
    pltpu.sync_copy(tmpv, ppr.at[pl.ds(base, B_PER_W), :])


TC_BLK = 512
TC_STEPS = BATCH // TC_BLK


def _normalize_rows(x):
    n = jnp.sqrt(jnp.sum(x * x, axis=-1, keepdims=True))
    return x / jnp.maximum(n, 1e-12)


def _tc_body(su_b, si_f, si_b, u0_b, p0_b, upr_b, ppr_f, ppr_b, out_ref):
    r = pl.program_id(0)
    ue = _normalize_rows(su_b[...] * 0.25)
    pe_all = _normalize_rows(si_f[...] * 0.25)
    pe_b = _normalize_rows(si_b[...] * 0.25)

    ratings = lax.dot_general(ue, pe_all, (((1,), (1,)), ((), ())),
                              preferred_element_type=jnp.float32)
    den1 = jnp.sum(jnp.exp(ratings * (1.0 / TAU1)), axis=1)
    diag = jnp.sum(ue * pe_b, axis=1)
    margin = jnp.sum(upr_b[...] * ppr_b[...], axis=1)
    phi = 1.0 - 1.0 / (1.0 + jnp.exp(-margin))
    x = jnp.clip(diag, -1.0 + 1e-07, 1.0 - 1e-07)
    adj = x * jnp.cos(phi) - jnp.sqrt(1.0 - x * x) * jnp.sin(phi)
    l1 = jnp.sum(jnp.log(den1) - adj * (1.0 / TAU1))

    un = _normalize_rows(upr_b[...])
    pn_all = _normalize_rows(ppr_f[...])
    pn_b = _normalize_rows(ppr_b[...])
    rat2 = lax.dot_general(un, pn_all, (((1,), (1,)), ((), ())),
                           preferred_element_type=jnp.float32)
    den2 = jnp.sum(jnp.exp(rat2 * (1.0 / TAU2)), axis=1)
    diag2 = jnp.sum(un * pn_b, axis=1)
    l2 = jnp.sum(jnp.log(den2) - diag2 * (1.0 / TAU2))

    su0 = jnp.sum(u0_b[...] ** 2)
    sp0 = jnp.sum(p0_b[...] ** 2)
    sun = jnp.sum(un ** 2)
    spn = jnp.sum(pn_b ** 2)

    lane = lax.broadcasted_iota(jnp.int32, (1, 128), 1)
    parts = (l1, l2, su0, sp0, sun, spn)
    vec = jnp.zeros((1, 128), jnp.float32)
    for k, p in enumerate(parts):
        vec = vec + jnp.where(lane == k, p, 0.0)

    @pl.when(r == 0)
    def _():
        out_ref[...] = vec

    @pl.when(r > 0)
    def _():
        out_ref[...] = out_ref[...] + vec

    @pl.when(r == TC_STEPS - 1)
    def _():
        tot = out_ref[...]

        def pick(k):
            return jnp.sum(jnp.where(lane == k, tot, 0.0))

        s0, s1, s2, s3, s4, s5 = (pick(k) for k in range(6))
        loss1 = (1.0 - W_LAMBDA) / BATCH * s0
        loss2 = W_LAMBDA / BATCH * s1
        reg1 = (0.5 * s2 + BATCH * 0.5 * s3) / BATCH
        reg2 = (0.5 * s4 + BATCH * 0.5 * s5) / BATCH
        outs = (loss1, loss2, DECAY * (reg1 + reg2), DECAY * reg2, DECAY * reg1)
        fv = jnp.zeros((1, 128), jnp.float32)
        for k, p in enumerate(outs):
            fv = fv + jnp.where(lane == k, p, 0.0)
        out_ref[...] = fv


_blk = pl.BlockSpec((TC_BLK, EMB), lambda r: (r, 0))
_full = pl.BlockSpec((BATCH, EMB), lambda r: (0, 0))

_tc_loss = pl.pallas_call(
    _tc_body,
    grid=(TC_STEPS,),
    in_specs=[_blk, _full, _blk, _blk, _blk, _blk, _full, _blk],
    out_specs=pl.BlockSpec((1, 128), lambda r: (0, 0)),
    out_shape=jax.ShapeDtypeStruct((1, 128), jnp.float32),
)


def kernel(users, pos_items, users_pop, pos_items_pop, embed_user, embed_item,
           embed_user_pop, embed_item_pop, edge_src, edge_dst, edge_val):
    t0 = jnp.concatenate([embed_user, embed_item], axis=0)
    t1 = _layer(edge_src, edge_dst, edge_val, t0)
    t2 = _layer(edge_src, edge_dst, edge_val, t1)
    t3 = _layer(edge_src, edge_dst, edge_val, t2)
    su, si, u0, p0, upr, ppr = _batch_gather(
        t0, t1, t2, t3,
        users.astype(jnp.int32), pos_items.astype(jnp.int32),
        users_pop.astype(jnp.int32), pos_items_pop.astype(jnp.int32),
        embed_user_pop, embed_item_pop)
    out = _tc_loss(su, si, u0, p0, upr, ppr)
    return (out[0, 0], out[0, 1], out[0, 2], out[0, 3], out[0, 4])


# speedup vs baseline: 5.1497x; 5.1497x over previous
import functools

import jax
import jax.numpy as jnp
from jax import lax
from jax.experimental import pallas as pl
from jax.experimental.pallas import tpu as pltpu
from jax.experimental.pallas import tpu_sc as plsc

N_USERS = 50000
N_ITEMS = 50000
NTOT = N_USERS + N_ITEMS
EMB = 32
BATCH = 4096
TAU1 = 0.07
TAU2 = 0.1
W_LAMBDA = 0.5
DECAY = 0.0001

NC = 2
NS = 16
LANES = 16

E_PER_DIR = 800000
E_PER_TILE = E_PER_DIR // NS
CHUNK = 80
GROUPS = CHUNK // LANES
N_CHUNKS = E_PER_TILE // CHUNK
HALF = 50048
PAD = HALF - N_USERS
NTOT_P = 2 * HALF
ROWS_PER_TILE = HALF // NS
WB = 184
WB_STEPS = ROWS_PER_TILE // WB

_mesh = plsc.VectorSubcoreMesh(core_axis_name="c", subcore_axis_name="s")


@functools.partial(
    pl.kernel,
    out_type=jax.ShapeDtypeStruct((NTOT_P, EMB), jnp.float32),
    mesh=_mesh,
    scratch_types=[
        pltpu.VMEM((CHUNK,), jnp.int32),
        pltpu.VMEM((CHUNK,), jnp.int32),
        pltpu.VMEM((CHUNK,), jnp.float32),
        pltpu.VMEM((CHUNK, EMB), jnp.float32),
        pltpu.VMEM((WB, EMB), jnp.float32),
        pltpu.VMEM_SHARED((HALF, EMB), jnp.float32),
        pltpu.SemaphoreType.DMA,
    ],
    compiler_params=pltpu.CompilerParams(use_tc_tiling_on_sc=False),
)
def _layer(src, dst, val, tin, tout, idx_s, idx_d, valv, rows, zwb, acc, sem):
    c = lax.axis_index("c")
    s = lax.axis_index("s")
    zero16 = jnp.zeros((LANES,), jnp.float32)
    iota16 = lax.broadcasted_iota(jnp.int32, (LANES,), 0)

    def _zrow(i, carry):
        zwb[i, pl.ds(0, LANES)] = zero16
        zwb[i, pl.ds(LANES, LANES)] = zero16
        return carry

    lax.fori_loop(0, WB, _zrow, 0)
    row0 = s * ROWS_PER_TILE
    for k in range(WB_STEPS):
        pltpu.sync_copy(zwb, acc.at[pl.ds(row0 + k * WB, WB), :])
    plsc.subcore_barrier()

    dbase = jnp.where(c == 0, N_USERS, 0).astype(jnp.int32)
    sbase = jnp.where(c == 0, 0, PAD).astype(jnp.int32)
    ebase = c * E_PER_DIR + s * E_PER_TILE

    def _chunk(i, carry):
        off = ebase + i * CHUNK
        pltpu.sync_copy(src.at[pl.ds(off, CHUNK)], idx_s)
        pltpu.sync_copy(dst.at[pl.ds(off, CHUNK)], idx_d)
        pltpu.sync_copy(val.at[pl.ds(off, CHUNK)], valv)
        for g in range(GROUPS):
            gs = g * LANES
            idx_s[pl.ds(gs, LANES)] = idx_s[pl.ds(gs, LANES)] + sbase
            idx_d[pl.ds(gs, LANES)] = idx_d[pl.ds(gs, LANES)] - dbase
        pltpu.async_copy(tin.at[idx_s], rows, sem).wait()
        for g in range(GROUPS):
            v = valv[pl.ds(g * LANES, LANES)]
            for j in range(LANES):
                i = g * LANES + j
                vb = jnp.full((LANES,), v[j], jnp.float32)
                rows[i, pl.ds(0, LANES)] = rows[i, pl.ds(0, LANES)] * vb
                rows[i, pl.ds(LANES, LANES)] = rows[i, pl.ds(LANES, LANES)] * vb
        pltpu.sync_copy(rows, acc.at[idx_d], add=True)
        return carry

    lax.fori_loop(0, N_CHUNKS, _chunk, 0)
    plsc.subcore_barrier()

    obase = jnp.where(c == 0, HALF, 0) + row0
    for k in range(WB_STEPS):
        pltpu.sync_copy(acc.at[pl.ds(row0 + k * WB, WB), :], zwb)
        pltpu.sync_copy(zwb, tout.at[pl.ds(obase + k * WB, WB), :])


B_PER_W = BATCH // (NC * NS)


@functools.partial(
    pl.kernel,
    out_type=(
        jax.ShapeDtypeStruct((BATCH, EMB), jnp.float32),
        jax.ShapeDtypeStruct((BATCH, EMB), jnp.float32),
        jax.ShapeDtypeStruct((BATCH, EMB), jnp.float32),
        jax.ShapeDtypeStruct((BATCH, EMB), jnp.float32),
        jax.ShapeDtypeStruct((BATCH, EMB), jnp.float32),
        jax.ShapeDtypeStruct((BATCH, EMB), jnp.float32),
    ),
    mesh=_mesh,
    scratch_types=[
        pltpu.VMEM((B_PER_W,), jnp.int32),
        pltpu.VMEM((B_PER_W, EMB), jnp.float32),
        pltpu.VMEM((B_PER_W, EMB), jnp.float32),
        pltpu.SemaphoreType.DMA,
    ],
    compiler_params=pltpu.CompilerParams(use_tc_tiling_on_sc=False),
)
def _batch_gather(t0, t1, t2, t3, users, items, upop, ipop, eup, eip,
                  su, si, u0, p0, upr, ppr, idxv, accv, tmpv, sem):
    c = lax.axis_index("c")
    s = lax.axis_index("s")
    wid = c * NS + s
    base = wid * B_PER_W

    def _acc_add(i, carry):
        accv[i, pl.ds(0, LANES)] = accv[i, pl.ds(0, LANES)] + tmpv[i, pl.ds(0, LANES)]
        accv[i, pl.ds(LANES, LANES)] = (
            accv[i, pl.ds(LANES, LANES)] + tmpv[i, pl.ds(LANES, LANES)]
        )
        return carry

    def _sum4(out_sum, out_first):
        pltpu.async_copy(t0.at[idxv], accv, sem).wait()
        if out_first is not None:
            pltpu.sync_copy(accv, out_first.at[pl.ds(base, B_PER_W), :])
        for t in (t1, t2, t3):
            pltpu.async_copy(t.at[idxv], tmpv, sem).wait()
            lax.fori_loop(0, B_PER_W, _acc_add, 0)
        pltpu.sync_copy(accv, out_sum.at[pl.ds(base, B_PER_W), :])

    pltpu.sync_copy(users.at[pl.ds(base, B_PER_W)], idxv)
    _sum4(su, u0)

    pltpu.sync_copy(items.at[pl.ds(base, B_PER_W)], idxv)
    for g in range(B_PER_W // LANES):
        gs = g * LANES
        idxv[pl.ds(gs, LANES)] = idxv[pl.ds(gs, LANES)] + HALF
    _sum4(si, p0)

    pltpu.sync_copy(upop.at[pl.ds(base, B_PER_W)], idxv)
    pltpu.async_copy(eup.at[idxv], tmpv, sem).wait()
    pltpu.sync_copy(tmpv, upr.at[pl.ds(base, B_PER_W), :])
    pltpu.sync_copy(ipop.at[pl.ds(base, B_PER_W)], idxv)
    pltpu.async_copy(eip.at[idxv], tmpv, sem).wait()
    pltpu.sync_copy(tmpv, ppr.at[pl.ds(base, B_PER_W), :])


TC_BLK = 512
TC_STEPS = BATCH // TC_BLK


def _normalize_rows(x):
    n = jnp.sqrt(jnp.sum(x * x, axis=-1, keepdims=True))
    return x / jnp.maximum(n, 1e-12)


def _tc_body(su_b, si_f, si_b, u0_b, p0_b, upr_b, ppr_f, ppr_b, out_ref):
    r = pl.program_id(0)
    ue = _normalize_rows(su_b[...] * 0.25)
    pe_all = _normalize_rows(si_f[...] * 0.25)
    pe_b = _normalize_rows(si_b[...] * 0.25)

    ratings = lax.dot_general(ue, pe_all, (((1,), (1,)), ((), ())),
                              preferred_element_type=jnp.float32)
    den1 = jnp.sum(jnp.exp(ratings * (1.0 / TAU1)), axis=1)
    diag = jnp.sum(ue * pe_b, axis=1)
    margin = jnp.sum(upr_b[...] * ppr_b[...], axis=1)
    phi = 1.0 - 1.0 / (1.0 + jnp.exp(-margin))
    x = jnp.clip(diag, -1.0 + 1e-07, 1.0 - 1e-07)
    adj = x * jnp.cos(phi) - jnp.sqrt(1.0 - x * x) * jnp.sin(phi)
    l1 = jnp.sum(jnp.log(den1) - adj * (1.0 / TAU1))

    un = _normalize_rows(upr_b[...])
    pn_all = _normalize_rows(ppr_f[...])
    pn_b = _normalize_rows(ppr_b[...])
    rat2 = lax.dot_general(un, pn_all, (((1,), (1,)), ((), ())),
                           preferred_element_type=jnp.float32)
    den2 = jnp.sum(jnp.exp(rat2 * (1.0 / TAU2)), axis=1)
    diag2 = jnp.sum(un * pn_b, axis=1)
    l2 = jnp.sum(jnp.log(den2) - diag2 * (1.0 / TAU2))

    su0 = jnp.sum(u0_b[...] ** 2)
    sp0 = jnp.sum(p0_b[...] ** 2)
    sun = jnp.sum(un ** 2)
    spn = jnp.sum(pn_b ** 2)

    lane = lax.broadcasted_iota(jnp.int32, (1, 128), 1)
    parts = (l1, l2, su0, sp0, sun, spn)
    vec = jnp.zeros((1, 128), jnp.float32)
    for k, p in enumerate(parts):
        vec = vec + jnp.where(lane == k, p, 0.0)

    @pl.when(r == 0)
    def _():
        out_ref[...] = vec

    @pl.when(r > 0)
    def _():
        out_ref[...] = out_ref[...] + vec

    @pl.when(r == TC_STEPS - 1)
    def _():
        tot = out_ref[...]

        def pick(k):
            return jnp.sum(jnp.where(lane == k, tot, 0.0))

        s0, s1, s2, s3, s4, s5 = (pick(k) for k in range(6))
        loss1 = (1.0 - W_LAMBDA) / BATCH * s0
        loss2 = W_LAMBDA / BATCH * s1
        reg1 = (0.5 * s2 + BATCH * 0.5 * s3) / BATCH
        reg2 = (0.5 * s4 + BATCH * 0.5 * s5) / BATCH
        outs = (loss1, loss2, DECAY * (reg1 + reg2), DECAY * reg2, DECAY * reg1)
        fv = jnp.zeros((1, 128), jnp.float32)
        for k, p in enumerate(outs):
            fv = fv + jnp.where(lane == k, p, 0.0)
        out_ref[...] = fv


_blk = pl.BlockSpec((TC_BLK, EMB), lambda r: (r, 0))
_full = pl.BlockSpec((BATCH, EMB), lambda r: (0, 0))

_tc_loss = pl.pallas_call(
    _tc_body,
    grid=(TC_STEPS,),
    in_specs=[_blk, _full, _blk, _blk, _blk, _blk, _full, _blk],
    out_specs=pl.BlockSpec((1, 128), lambda r: (0, 0)),
    out_shape=jax.ShapeDtypeStruct((1, 128), jnp.float32),
)


def kernel(users, pos_items, users_pop, pos_items_pop, embed_user, embed_item,
           embed_user_pop, embed_item_pop, edge_src, edge_dst, edge_val):
    pad = jnp.zeros((PAD, EMB), jnp.float32)
    t0 = jnp.concatenate([embed_user, pad, embed_item, pad], axis=0)
    t1 = _layer(edge_src, edge_dst, edge_val, t0)
    t2 = _layer(edge_src, edge_dst, edge_val, t1)
    t3 = _layer(edge_src, edge_dst, edge_val, t2)
    su, si, u0, p0, upr, ppr = _batch_gather(
        t0, t1, t2, t3,
        users.astype(jnp.int32), pos_items.astype(jnp.int32),
        users_pop.astype(jnp.int32), pos_items_pop.astype(jnp.int32),
        embed_user_pop, embed_item_pop)
    out = _tc_loss(su, si, si, u0, p0, upr, ppr, ppr)
    return (out[0, 0], out[0, 1], out[0, 2], out[0, 3], out[0, 4])


# speedup vs baseline: 16.9648x; 3.2943x over previous
import functools

import jax
import jax.numpy as jnp
from jax import lax
from jax.experimental import pallas as pl
from jax.experimental.pallas import tpu as pltpu
from jax.experimental.pallas import tpu_sc as plsc

N_USERS = 50000
N_ITEMS = 50000
NTOT = N_USERS + N_ITEMS
EMB = 32
BATCH = 4096
TAU1 = 0.07
TAU2 = 0.1
W_LAMBDA = 0.5
DECAY = 0.0001

NC = 2
NS = 16
LANES = 16

E_PER_DIR = 800000
E_PER_TILE = E_PER_DIR // NS
SUB = 80
CHUNK = SUB
N_CHUNKS = E_PER_TILE // CHUNK
HALF = 50048
PAD = HALF - N_USERS
NTOT_P = 2 * HALF
ROWS_PER_TILE = HALF // NS
WB = 136
WB_STEPS = ROWS_PER_TILE // WB

_mesh = plsc.VectorSubcoreMesh(core_axis_name="c", subcore_axis_name="s")


@functools.partial(
    pl.kernel,
    out_type=jax.ShapeDtypeStruct((NTOT_P, EMB), jnp.float32),
    mesh=_mesh,
    scratch_types=[
        pltpu.VMEM((4, CHUNK), jnp.int32),
        pltpu.VMEM((4, SUB), jnp.int32),
        pltpu.VMEM((4, CHUNK), jnp.float32),
        pltpu.VMEM((CHUNK, EMB), jnp.float32),
        pltpu.VMEM((CHUNK, EMB), jnp.float32),
        pltpu.VMEM((WB, EMB), jnp.float32),
        pltpu.VMEM_SHARED((HALF, EMB), jnp.float32),
        pltpu.SemaphoreType.DMA,
        pltpu.SemaphoreType.DMA,
        pltpu.SemaphoreType.DMA,
        pltpu.SemaphoreType.DMA,
        pltpu.SemaphoreType.DMA,
    ],
    compiler_params=pltpu.CompilerParams(use_tc_tiling_on_sc=False),
)
def _layer(src, dst2, val, tin, tout, idx_s, idx_d, valv, rows0, rows1,
           zwb, acc, esemA, esemB, gsem0, gsem1, ssem):
    rows = (rows0, rows1)
    esem = (esemA, esemB)
    gsem = (gsem0, gsem1)
    c = lax.axis_index("c")
    s = lax.axis_index("s")
    zero16 = jnp.zeros((LANES,), jnp.float32)

    def _zrow(i, carry):
        zwb[i, pl.ds(0, LANES)] = zero16
        zwb[i, pl.ds(LANES, LANES)] = zero16
        return carry

    lax.fori_loop(0, WB, _zrow, 0)
    row0 = s * ROWS_PER_TILE
    for k in range(WB_STEPS):
        pltpu.sync_copy(zwb, acc.at[pl.ds(row0 + k * WB, WB), :])
    plsc.subcore_barrier()

    dbase = jnp.where(c == 0, N_USERS, 0).astype(jnp.int32)
    sbase = jnp.where(c == 0, 0, PAD).astype(jnp.int32)
    ebase = c * E_PER_DIR + s * E_PER_TILE

    rbase = ebase // SUB

    def _edge_fire(i, e, p):
        off = ebase + i * CHUNK
        pltpu.async_copy(src.at[pl.ds(off, CHUNK)], idx_s.at[e], esem[p])
        pltpu.async_copy(val.at[pl.ds(off, CHUNK)], valv.at[e], esem[p])
        pltpu.async_copy(dst2.at[pl.ds(rbase + i, 1), :],
                         idx_d.at[pl.ds(e, 1), :], esem[p])

    def _edge_wait(i, e, p):
        off = ebase + i * CHUNK
        pltpu.make_async_copy(src.at[pl.ds(off, CHUNK)], idx_s.at[e],
                              esem[p]).wait()
        pltpu.make_async_copy(val.at[pl.ds(off, CHUNK)], valv.at[e],
                              esem[p]).wait()
        pltpu.make_async_copy(dst2.at[pl.ds(rbase + i, 1), :],
                              idx_d.at[pl.ds(e, 1), :], esem[p]).wait()

    def _adjust(e):
        def _adj(g, carry):
            gs = g * LANES
            idx_s[e, pl.ds(gs, LANES)] = idx_s[e, pl.ds(gs, LANES)] + sbase
            idx_d[e, pl.ds(gs, LANES)] = idx_d[e, pl.ds(gs, LANES)] - dbase
            return carry

        lax.fori_loop(0, CHUNK // LANES, _adj, 0)

    def _gather_fire(e, rb):
        return pltpu.async_copy(tin.at[idx_s.at[e]], rows[rb], gsem[rb])

    def _scale(e, rb):
        def _sc(g, carry):
            v = valv[e, pl.ds(g * LANES, LANES)]
            base_row = g * LANES
            for jj in range(LANES):
                rr = base_row + jj
                vb = jnp.full((LANES,), v[jj], jnp.float32)
                rows[rb][rr, pl.ds(0, LANES)] = (
                    rows[rb][rr, pl.ds(0, LANES)] * vb)
                rows[rb][rr, pl.ds(LANES, LANES)] = (
                    rows[rb][rr, pl.ds(LANES, LANES)] * vb)
            return carry

        lax.fori_loop(0, CHUNK // LANES, _sc, 0)

    def _scatter_fire(e, rb):
        return pltpu.async_copy(rows[rb], acc.at[idx_d.at[e]], ssem, add=True)

    _edge_fire(0, 0, 0)
    _edge_fire(1, 1, 1)

    def _body(k, carry):
        i = k * 2
        e0 = i & 3
        e1 = (i + 1) & 3
        _edge_wait(i, e0, 0)
        _adjust(e0)
        gd0 = _gather_fire(e0, 0)
        _edge_wait(i + 1, e1, 1)
        _adjust(e1)
        gd1 = _gather_fire(e1, 1)
        _edge_fire(i + 2, (e0 + 2) & 3, 0)

        @pl.when(i + 3 < N_CHUNKS)
        def _():
            _edge_fire(i + 3, (e1 + 2) & 3, 1)

        gd0.wait()
        _scale(e0, 0)
        sd0 = _scatter_fire(e0, 0)
        gd1.wait()
        _scale(e1, 1)
        sd1 = _scatter_fire(e1, 1)
        sd0.wait()
        sd1.wait()
        return carry

    lax.fori_loop(0, N_CHUNKS // 2, _body, 0)
    _edge_wait(N_CHUNKS - 1, 0, 0)
    _adjust(0)
    gd = _gather_fire(0, 0)
    gd.wait()
    _scale(0, 0)
    _scatter_fire(0, 0).wait()
    plsc.subcore_barrier()

    obase = jnp.where(c == 0, HALF, 0) + row0
    for k in range(WB_STEPS):
        pltpu.sync_copy(acc.at[pl.ds(row0 + k * WB, WB), :], zwb)
        pltpu.sync_copy(zwb, tout.at[pl.ds(obase + k * WB, WB), :])


B_PER_W = BATCH // (NC * NS)


@functools.partial(
    pl.kernel,
    out_type=(
        jax.ShapeDtypeStruct((BATCH, EMB), jnp.float32),
        jax.ShapeDtypeStruct((BATCH, EMB), jnp.float32),
        jax.ShapeDtypeStruct((BATCH, EMB), jnp.float32),
        jax.ShapeDtypeStruct((BATCH, EMB), jnp.float32),
        jax.ShapeDtypeStruct((BATCH, EMB), jnp.float32),
        jax.ShapeDtypeStruct((BATCH, EMB), jnp.float32),
    ),
    mesh=_mesh,
    scratch_types=[
        pltpu.VMEM((B_PER_W,), jnp.int32),
        pltpu.VMEM((B_PER_W, EMB), jnp.float32),
        pltpu.VMEM((B_PER_W, EMB), jnp.float32),
        pltpu.SemaphoreType.DMA,
    ],
    compiler_params=pltpu.CompilerParams(use_tc_tiling_on_sc=False),
)
def _batch_gather(t0, t1, t2, t3, users, items, upop, ipop, eup, eip,
                  su, si, u0, p0, upr, ppr, idxv, accv, tmpv, sem):
    c = lax.axis_index("c")
    s = lax.axis_index("s")
    wid = c * NS + s
    base = wid * B_PER_W

    def _acc_add(i, carry):
        accv[i, pl.ds(0, LANES)] = accv[i, pl.ds(0, LANES)] + tmpv[i, pl.ds(0, LANES)]
        accv[i, pl.ds(LANES, LANES)] = (
            accv[i, pl.ds(LANES, LANES)] + tmpv[i, pl.ds(LANES, LANES)]
        )
        return carry

    def _sum4(out_sum, out_first):
        pltpu.async_copy(t0.at[idxv], accv, sem).wait()
        if out_first is not None:
            pltpu.sync_copy(accv, out_first.at[pl.ds(base, B_PER_W), :])
        for t in (t1, t2, t3):
            pltpu.async_copy(t.at[idxv], tmpv, sem).wait()
            lax.fori_loop(0, B_PER_W, _acc_add, 0)
        pltpu.sync_copy(accv, out_sum.at[pl.ds(base, B_PER_W), :])

    pltpu.sync_copy(users.at[pl.ds(base, B_PER_W)], idxv)
    _sum4(su, u0)

    pltpu.sync_copy(items.at[pl.ds(base, B_PER_W)], idxv)
    for g in range(B_PER_W // LANES):
        gs = g * LANES
        idxv[pl.ds(gs, LANES)] = idxv[pl.ds(gs, LANES)] + HALF
    _sum4(si, p0)

    pltpu.sync_copy(upop.at[pl.ds(base, B_PER_W)], idxv)
    pltpu.async_copy(eup.at[idxv], tmpv, sem).wait()
    pltpu.sync_copy(tmpv, upr.at[pl.ds(base, B_PER_W), :])
    pltpu.sync_copy(ipop.at[pl.ds(base, B_PER_W)], idxv)
    pltpu.async_copy(eip.at[idxv], tmpv, sem).wait()
    pltpu.sync_copy(tmpv, ppr.at[pl.ds(base, B_PER_W), :])


TC_BLK = 512
TC_STEPS = BATCH // TC_BLK


def _normalize_rows(x):
    n = jnp.sqrt(jnp.sum(x * x, axis=-1, keepdims=True))
    return x / jnp.maximum(n, 1e-12)


def _tc_body(su_b, si_f, si_b, u0_b, p0_b, upr_b, ppr_f, ppr_b, out_ref):
    r = pl.program_id(0)
    ue = _normalize_rows(su_b[...] * 0.25)
    pe_all = _normalize_rows(si_f[...] * 0.25)
    pe_b = _normalize_rows(si_b[...] * 0.25)

    ratings = lax.dot_general(ue, pe_all, (((1,), (1,)), ((), ())),
                              preferred_element_type=jnp.float32)
    den1 = jnp.sum(jnp.exp(ratings * (1.0 / TAU1)), axis=1)
    diag = jnp.sum(ue * pe_b, axis=1)
    margin = jnp.sum(upr_b[...] * ppr_b[...], axis=1)
    phi = 1.0 - 1.0 / (1.0 + jnp.exp(-margin))
    x = jnp.clip(diag, -1.0 + 1e-07, 1.0 - 1e-07)
    adj = x * jnp.cos(phi) - jnp.sqrt(1.0 - x * x) * jnp.sin(phi)
    l1 = jnp.sum(jnp.log(den1) - adj * (1.0 / TAU1))

    un = _normalize_rows(upr_b[...])
    pn_all = _normalize_rows(ppr_f[...])
    pn_b = _normalize_rows(ppr_b[...])
    rat2 = lax.dot_general(un, pn_all, (((1,), (1,)), ((), ())),
                           preferred_element_type=jnp.float32)
    den2 = jnp.sum(jnp.exp(rat2 * (1.0 / TAU2)), axis=1)
    diag2 = jnp.sum(un * pn_b, axis=1)
    l2 = jnp.sum(jnp.log(den2) - diag2 * (1.0 / TAU2))

    su0 = jnp.sum(u0_b[...] ** 2)
    sp0 = jnp.sum(p0_b[...] ** 2)
    sun = jnp.sum(un ** 2)
    spn = jnp.sum(pn_b ** 2)

    lane = lax.broadcasted_iota(jnp.int32, (1, 128), 1)
    parts = (l1, l2, su0, sp0, sun, spn)
    vec = jnp.zeros((1, 128), jnp.float32)
    for k, p in enumerate(parts):
        vec = vec + jnp.where(lane == k, p, 0.0)

    @pl.when(r == 0)
    def _():
        out_ref[...] = vec

    @pl.when(r > 0)
    def _():
        out_ref[...] = out_ref[...] + vec

    @pl.when(r == TC_STEPS - 1)
    def _():
        tot = out_ref[...]

        def pick(k):
            return jnp.sum(jnp.where(lane == k, tot, 0.0))

        s0, s1, s2, s3, s4, s5 = (pick(k) for k in range(6))
        loss1 = (1.0 - W_LAMBDA) / BATCH * s0
        loss2 = W_LAMBDA / BATCH * s1
        reg1 = (0.5 * s2 + BATCH * 0.5 * s3) / BATCH
        reg2 = (0.5 * s4 + BATCH * 0.5 * s5) / BATCH
        outs = (loss1, loss2, DECAY * (reg1 + reg2), DECAY * reg2, DECAY * reg1)
        fv = jnp.zeros((1, 128), jnp.float32)
        for k, p in enumerate(outs):
            fv = fv + jnp.where(lane == k, p, 0.0)
        out_ref[...] = fv


_blk = pl.BlockSpec((TC_BLK, EMB), lambda r: (r, 0))
_full = pl.BlockSpec((BATCH, EMB), lambda r: (0, 0))

_tc_loss = pl.pallas_call(
    _tc_body,
    grid=(TC_STEPS,),
    in_specs=[_blk, _full, _blk, _blk, _blk, _blk, _full, _blk],
    out_specs=pl.BlockSpec((1, 128), lambda r: (0, 0)),
    out_shape=jax.ShapeDtypeStruct((1, 128), jnp.float32),
)


def kernel(users, pos_items, users_pop, pos_items_pop, embed_user, embed_item,
           embed_user_pop, embed_item_pop, edge_src, edge_dst, edge_val):
    pad = jnp.zeros((PAD, EMB), jnp.float32)
    t0 = jnp.concatenate([embed_user, pad, embed_item, pad], axis=0)
    dst2 = edge_dst.reshape(-1, SUB)
    t1 = _layer(edge_src, dst2, edge_val, t0)
    t2 = _layer(edge_src, dst2, edge_val, t1)
    t3 = _layer(edge_src, dst2, edge_val, t2)
    su, si, u0, p0, upr, ppr = _batch_gather(
        t0, t1, t2, t3,
        users.astype(jnp.int32), pos_items.astype(jnp.int32),
        users_pop.astype(jnp.int32), pos_items_pop.astype(jnp.int32),
        embed_user_pop, embed_item_pop)
    out = _tc_loss(su, si, si, u0, p0, upr, ppr, ppr)
    return (out[0, 0], out[0, 1], out[0, 2], out[0, 3], out[0, 4])


# speedup vs baseline: 22.6611x; 1.3358x over previous
import functools

import jax
import jax.numpy as jnp
from jax import lax
from jax.experimental import pallas as pl
from jax.experimental.pallas import tpu as pltpu
from jax.experimental.pallas import tpu_sc as plsc

N_USERS = 50000
N_ITEMS = 50000
NTOT = N_USERS + N_ITEMS
EMB = 32
BATCH = 4096
TAU1 = 0.07
TAU2 = 0.1
W_LAMBDA = 0.5
DECAY = 0.0001

NC = 2
NS = 16
LANES = 16

E_PER_DIR = 800000
E_PER_TILE = E_PER_DIR // NS
SUB = 80
CHUNK = SUB
N_CHUNKS = E_PER_TILE // CHUNK
HALF = 50048
PAD = HALF - N_USERS
NTOT_P = 2 * HALF
ROWS_PER_TILE = HALF // NS
WB = 136
WB_STEPS = ROWS_PER_TILE // WB

_mesh = plsc.VectorSubcoreMesh(core_axis_name="c", subcore_axis_name="s")


@functools.partial(
    pl.kernel,
    out_type=jax.ShapeDtypeStruct((NTOT_P, EMB), jnp.float32),
    mesh=_mesh,
    scratch_types=[
        pltpu.VMEM((8, CHUNK), jnp.int32),
        pltpu.VMEM((8, SUB), jnp.int32),
        pltpu.VMEM((8, CHUNK), jnp.float32),
        pltpu.VMEM((CHUNK, EMB), jnp.float32),
        pltpu.VMEM((CHUNK, EMB), jnp.float32),
        pltpu.VMEM((CHUNK, EMB), jnp.float32),
        pltpu.VMEM((CHUNK, EMB), jnp.float32),
        pltpu.VMEM((WB, EMB), jnp.float32),
        pltpu.VMEM_SHARED((HALF, EMB), jnp.float32),
        pltpu.SemaphoreType.DMA,
        pltpu.SemaphoreType.DMA,
        pltpu.SemaphoreType.DMA,
        pltpu.SemaphoreType.DMA,
        pltpu.SemaphoreType.DMA,
        pltpu.SemaphoreType.DMA,
    ],
    compiler_params=pltpu.CompilerParams(use_tc_tiling_on_sc=False),
)
def _layer(src, dst2, val, tin, tout, idx_s, idx_d, valv,
           rows0, rows1, rows2, rows3, zwb, acc,
           esem, gsem0, gsem1, gsem2, gsem3, ssem):
    rows = (rows0, rows1, rows2, rows3)
    gsem = (gsem0, gsem1, gsem2, gsem3)
    c = lax.axis_index("c")
    s = lax.axis_index("s")
    zero16 = jnp.zeros((LANES,), jnp.float32)

    def _zrow(i, carry):
        zwb[i, pl.ds(0, LANES)] = zero16
        zwb[i, pl.ds(LANES, LANES)] = zero16
        return carry

    lax.fori_loop(0, WB, _zrow, 0)
    row0 = s * ROWS_PER_TILE
    for k in range(WB_STEPS):
        pltpu.sync_copy(zwb, acc.at[pl.ds(row0 + k * WB, WB), :])
    plsc.subcore_barrier()

    dbase = jnp.where(c == 0, N_USERS, 0).astype(jnp.int32)
    sbase = jnp.where(c == 0, 0, PAD).astype(jnp.int32)
    ebase = c * E_PER_DIR + s * E_PER_TILE

    rbase = ebase // SUB

    def _edge_fire(i, e):
        off = ebase + i * CHUNK
        pltpu.async_copy(src.at[pl.ds(off, CHUNK)], idx_s.at[e], esem)
        pltpu.async_copy(val.at[pl.ds(off, CHUNK)], valv.at[e], esem)
        pltpu.async_copy(dst2.at[pl.ds(rbase + i, 1), :],
                         idx_d.at[pl.ds(e, 1), :], esem)

    def _edge_wait(i, e):
        off = ebase + i * CHUNK
        pltpu.make_async_copy(src.at[pl.ds(off, CHUNK)], idx_s.at[e],
                              esem).wait()
        pltpu.make_async_copy(val.at[pl.ds(off, CHUNK)], valv.at[e],
                              esem).wait()
        pltpu.make_async_copy(dst2.at[pl.ds(rbase + i, 1), :],
                              idx_d.at[pl.ds(e, 1), :], esem).wait()

    def _adjust(e):
        def _adj(g, carry):
            gs = g * LANES
            idx_s[e, pl.ds(gs, LANES)] = idx_s[e, pl.ds(gs, LANES)] + sbase
            idx_d[e, pl.ds(gs, LANES)] = idx_d[e, pl.ds(gs, LANES)] - dbase
            return carry

        lax.fori_loop(0, CHUNK // LANES, _adj, 0)

    def _gather_fire(e, rb):
        return pltpu.async_copy(tin.at[idx_s.at[e]], rows[rb], gsem[rb])

    def _scale(e, rb):
        def _sc(g, carry):
            v = valv[e, pl.ds(g * LANES, LANES)]
            base_row = g * LANES
            for jj in range(LANES):
                rr = base_row + jj
                vb = jnp.full((LANES,), v[jj], jnp.float32)
                rows[rb][rr, pl.ds(0, LANES)] = (
                    rows[rb][rr, pl.ds(0, LANES)] * vb)
                rows[rb][rr, pl.ds(LANES, LANES)] = (
                    rows[rb][rr, pl.ds(LANES, LANES)] * vb)
            return carry

        lax.fori_loop(0, CHUNK // LANES, _sc, 0)

    def _scatter_fire(e, rb):
        return pltpu.async_copy(rows[rb], acc.at[idx_d.at[e]], ssem, add=True)

    for t in range(4):
        _edge_fire(t, t)

    def _body(k, carry):
        i = k * 4
        ib = i & 4
        nb = 4 - ib
        for t in range(4):
            _edge_wait(i + t, ib + t)
        gds = []
        for t in range(4):
            _adjust(ib + t)
            gds.append(_gather_fire(ib + t, t))
        for t in range(4):
            @pl.when(i + 4 + t < N_CHUNKS)
            def _(t=t):
                _edge_fire(i + 4 + t, nb + t)
        sds = []
        for t in range(4):
            gds[t].wait()
            _scale(ib + t, t)
            sds.append(_scatter_fire(ib + t, t))
        for d in sds:
            d.wait()
        return carry

    lax.fori_loop(0, N_CHUNKS // 4, _body, 0)
    _edge_wait(N_CHUNKS - 1, 0)
    _adjust(0)
    gd = _gather_fire(0, 0)
    gd.wait()
    _scale(0, 0)
    _scatter_fire(0, 0).wait()
    plsc.subcore_barrier()

    obase = jnp.where(c == 0, HALF, 0) + row0
    for k in range(WB_STEPS):
        pltpu.sync_copy(acc.at[pl.ds(row0 + k * WB, WB), :], zwb)
        pltpu.sync_copy(zwb, tout.at[pl.ds(obase + k * WB, WB), :])


B_PER_W = BATCH // (NC * NS)


@functools.partial(
    pl.kernel,
    out_type=(
        jax.ShapeDtypeStruct((BATCH, EMB), jnp.float32),
        jax.ShapeDtypeStruct((BATCH, EMB), jnp.float32),
        jax.ShapeDtypeStruct((BATCH, EMB), jnp.float32),
        jax.ShapeDtypeStruct((BATCH, EMB), jnp.float32),
        jax.ShapeDtypeStruct((BATCH, EMB), jnp.float32),
        jax.ShapeDtypeStruct((BATCH, EMB), jnp.float32),
    ),
    mesh=_mesh,
    scratch_types=[
        pltpu.VMEM((B_PER_W,), jnp.int32),
        pltpu.VMEM((B_PER_W, EMB), jnp.float32),
        pltpu.VMEM((B_PER_W, EMB), jnp.float32),
        pltpu.SemaphoreType.DMA,
    ],
    compiler_params=pltpu.CompilerParams(use_tc_tiling_on_sc=False),
)
def _batch_gather(t0, t1, t2, t3, users, items, upop, ipop, eup, eip,
                  su, si, u0, p0, upr, ppr, idxv, accv, tmpv, sem):
    c = lax.axis_index("c")
    s = lax.axis_index("s")
    wid = c * NS + s
    base = wid * B_PER_W

    def _acc_add(i, carry):
        accv[i, pl.ds(0, LANES)] = accv[i, pl.ds(0, LANES)] + tmpv[i, pl.ds(0, LANES)]
        accv[i, pl.ds(LANES, LANES)] = (
            accv[i, pl.ds(LANES, LANES)] + tmpv[i, pl.ds(LANES, LANES)]
        )
        return carry

    def _sum4(out_sum, out_first):
        pltpu.async_copy(t0.at[idxv], accv, sem).wait()
        if out_first is not None:
            pltpu.sync_copy(accv, out_first.at[pl.ds(base, B_PER_W), :])
        for t in (t1, t2, t3):
            pltpu.async_copy(t.at[idxv], tmpv, sem).wait()
            lax.fori_loop(0, B_PER_W, _acc_add, 0)
        pltpu.sync_copy(accv, out_sum.at[pl.ds(base, B_PER_W), :])

    pltpu.sync_copy(users.at[pl.ds(base, B_PER_W)], idxv)
    _sum4(su, u0)

    pltpu.sync_copy(items.at[pl.ds(base, B_PER_W)], idxv)
    for g in range(B_PER_W // LANES):
        gs = g * LANES
        idxv[pl.ds(gs, LANES)] = idxv[pl.ds(gs, LANES)] + HALF
    _sum4(si, p0)

    pltpu.sync_copy(upop.at[pl.ds(base, B_PER_W)], idxv)
    pltpu.async_copy(eup.at[idxv], tmpv, sem).wait()
    pltpu.sync_copy(tmpv, upr.at[pl.ds(base, B_PER_W), :])
    pltpu.sync_copy(ipop.at[pl.ds(base, B_PER_W)], idxv)
    pltpu.async_copy(eip.at[idxv], tmpv, sem).wait()
    pltpu.sync_copy(tmpv, ppr.at[pl.ds(base, B_PER_W), :])


TC_BLK = 512
TC_STEPS = BATCH // TC_BLK


def _normalize_rows(x):
    n = jnp.sqrt(jnp.sum(x * x, axis=-1, keepdims=True))
    return x / jnp.maximum(n, 1e-12)


def _tc_body(su_b, si_f, si_b, u0_b, p0_b, upr_b, ppr_f, ppr_b, out_ref):
    r = pl.program_id(0)
    ue = _normalize_rows(su_b[...] * 0.25)
    pe_all = _normalize_rows(si_f[...] * 0.25)
    pe_b = _normalize_rows(si_b[...] * 0.25)

    ratings = lax.dot_general(ue, pe_all, (((1,), (1,)), ((), ())),
                              preferred_element_type=jnp.float32)
    den1 = jnp.sum(jnp.exp(ratings * (1.0 / TAU1)), axis=1)
    diag = jnp.sum(ue * pe_b, axis=1)
    margin = jnp.sum(upr_b[...] * ppr_b[...], axis=1)
    phi = 1.0 - 1.0 / (1.0 + jnp.exp(-margin))
    x = jnp.clip(diag, -1.0 + 1e-07, 1.0 - 1e-07)
    adj = x * jnp.cos(phi) - jnp.sqrt(1.0 - x * x) * jnp.sin(phi)
    l1 = jnp.sum(jnp.log(den1) - adj * (1.0 / TAU1))

    un = _normalize_rows(upr_b[...])
    pn_all = _normalize_rows(ppr_f[...])
    pn_b = _normalize_rows(ppr_b[...])
    rat2 = lax.dot_general(un, pn_all, (((1,), (1,)), ((), ())),
                           preferred_element_type=jnp.float32)
    den2 = jnp.sum(jnp.exp(rat2 * (1.0 / TAU2)), axis=1)
    diag2 = jnp.sum(un * pn_b, axis=1)
    l2 = jnp.sum(jnp.log(den2) - diag2 * (1.0 / TAU2))

    su0 = jnp.sum(u0_b[...] ** 2)
    sp0 = jnp.sum(p0_b[...] ** 2)
    sun = jnp.sum(un ** 2)
    spn = jnp.sum(pn_b ** 2)

    lane = lax.broadcasted_iota(jnp.int32, (1, 128), 1)
    parts = (l1, l2, su0, sp0, sun, spn)
    vec = jnp.zeros((1, 128), jnp.float32)
    for k, p in enumerate(parts):
        vec = vec + jnp.where(lane == k, p, 0.0)

    @pl.when(r == 0)
    def _():
        out_ref[...] = vec

    @pl.when(r > 0)
    def _():
        out_ref[...] = out_ref[...] + vec

    @pl.when(r == TC_STEPS - 1)
    def _():
        tot = out_ref[...]

        def pick(k):
            return jnp.sum(jnp.where(lane == k, tot, 0.0))

        s0, s1, s2, s3, s4, s5 = (pick(k) for k in range(6))
        loss1 = (1.0 - W_LAMBDA) / BATCH * s0
        loss2 = W_LAMBDA / BATCH * s1
        reg1 = (0.5 * s2 + BATCH * 0.5 * s3) / BATCH
        reg2 = (0.5 * s4 + BATCH * 0.5 * s5) / BATCH
        outs = (loss1, loss2, DECAY * (reg1 + reg2), DECAY * reg2, DECAY * reg1)
        fv = jnp.zeros((1, 128), jnp.float32)
        for k, p in enumerate(outs):
            fv = fv + jnp.where(lane == k, p, 0.0)
        out_ref[...] = fv


_blk = pl.BlockSpec((TC_BLK, EMB), lambda r: (r, 0))
_full = pl.BlockSpec((BATCH, EMB), lambda r: (0, 0))

_tc_loss = pl.pallas_call(
    _tc_body,
    grid=(TC_STEPS,),
    in_specs=[_blk, _full, _blk, _blk, _blk, _blk, _full, _blk],
    out_specs=pl.BlockSpec((1, 128), lambda r: (0, 0)),
    out_shape=jax.ShapeDtypeStruct((1, 128), jnp.float32),
)


def kernel(users, pos_items, users_pop, pos_items_pop, embed_user, embed_item,
           embed_user_pop, embed_item_pop, edge_src, edge_dst, edge_val):
    pad = jnp.zeros((PAD, EMB), jnp.float32)
    t0 = jnp.concatenate([embed_user, pad, embed_item, pad], axis=0)
    dst2 = edge_dst.reshape(-1, SUB)
    t1 = _layer(edge_src, dst2, edge_val, t0)
    t2 = _layer(edge_src, dst2, edge_val, t1)
    t3 = _layer(edge_src, dst2, edge_val, t2)
    su, si, u0, p0, upr, ppr = _batch_gather(
        t0, t1, t2, t3,
        users.astype(jnp.int32), pos_items.astype(jnp.int32),
        users_pop.astype(jnp.int32), pos_items_pop.astype(jnp.int32),
        embed_user_pop, embed_item_pop)
    out = _tc_loss(su, si, si, u0, p0, upr, ppr, ppr)
    return (out[0, 0], out[0, 1], out[0, 2], out[0, 3], out[0, 4])


# speedup vs baseline: 22.9765x; 1.0139x over previous
import functools

import jax
import jax.numpy as jnp
from jax import lax
from jax.experimental import pallas as pl
from jax.experimental.pallas import tpu as pltpu
from jax.experimental.pallas import tpu_sc as plsc

N_USERS = 50000
N_ITEMS = 50000
NTOT = N_USERS + N_ITEMS
EMB = 32
BATCH = 4096
TAU1 = 0.07
TAU2 = 0.1
W_LAMBDA = 0.5
DECAY = 0.0001

NC = 2
NS = 16
LANES = 16

E_PER_DIR = 800000
E_PER_TILE = E_PER_DIR // NS
SUB = 80
CHUNK = SUB
N_CHUNKS = E_PER_TILE // CHUNK
HALF = 50048
PAD = HALF - N_USERS
NTOT_P = 2 * HALF
ROWS_PER_TILE = HALF // NS
WB = 136
WB_STEPS = ROWS_PER_TILE // WB

_mesh = plsc.VectorSubcoreMesh(core_axis_name="c", subcore_axis_name="s")


@functools.partial(
    pl.kernel,
    out_type=jax.ShapeDtypeStruct((NTOT_P, EMB), jnp.float32),
    mesh=_mesh,
    scratch_types=[
        pltpu.VMEM((8, CHUNK), jnp.int32),
        pltpu.VMEM((8, SUB), jnp.int32),
        pltpu.VMEM((8, CHUNK), jnp.float32),
        pltpu.VMEM((CHUNK, EMB), jnp.float32),
        pltpu.VMEM((CHUNK, EMB), jnp.float32),
        pltpu.VMEM((CHUNK, EMB), jnp.float32),
        pltpu.VMEM((CHUNK, EMB), jnp.float32),
        pltpu.VMEM((WB, EMB), jnp.float32),
        pltpu.VMEM_SHARED((HALF, EMB), jnp.float32),
        pltpu.SemaphoreType.DMA,
        pltpu.SemaphoreType.DMA,
        pltpu.SemaphoreType.DMA,
        pltpu.SemaphoreType.DMA,
        pltpu.SemaphoreType.DMA,
        pltpu.SemaphoreType.DMA,
    ],
    compiler_params=pltpu.CompilerParams(use_tc_tiling_on_sc=False),
)
def _layer(src, dst2, val, tin, tout, idx_s, idx_d, valv,
           rows0, rows1, rows2, rows3, zwb, acc,
           esem, gsem0, gsem1, gsem2, gsem3, ssem):
    rows = (rows0, rows1, rows2, rows3)
    gsem = (gsem0, gsem1, gsem2, gsem3)
    c = lax.axis_index("c")
    s = lax.axis_index("s")
    zero16 = jnp.zeros((LANES,), jnp.float32)

    def _zrow(i, carry):
        zwb[i, pl.ds(0, LANES)] = zero16
        zwb[i, pl.ds(LANES, LANES)] = zero16
        return carry

    lax.fori_loop(0, WB, _zrow, 0)
    row0 = s * ROWS_PER_TILE
    zds = [pltpu.async_copy(zwb, acc.at[pl.ds(row0 + k * WB, WB), :], gsem0)
           for k in range(WB_STEPS)]
    for d in zds:
        d.wait()
    plsc.subcore_barrier()

    dbase = jnp.where(c == 0, N_USERS, 0).astype(jnp.int32)
    sbase = jnp.where(c == 0, 0, PAD).astype(jnp.int32)
    ebase = c * E_PER_DIR + s * E_PER_TILE

    rbase = ebase // SUB

    def _edge_fire(i, e):
        off = ebase + i * CHUNK
        pltpu.async_copy(src.at[pl.ds(off, CHUNK)], idx_s.at[e], esem)
        pltpu.async_copy(val.at[pl.ds(off, CHUNK)], valv.at[e], esem)
        pltpu.async_copy(dst2.at[pl.ds(rbase + i, 1), :],
                         idx_d.at[pl.ds(e, 1), :], esem)

    def _edge_wait(i, e):
        off = ebase + i * CHUNK
        pltpu.make_async_copy(src.at[pl.ds(off, CHUNK)], idx_s.at[e],
                              esem).wait()
        pltpu.make_async_copy(val.at[pl.ds(off, CHUNK)], valv.at[e],
                              esem).wait()
        pltpu.make_async_copy(dst2.at[pl.ds(rbase + i, 1), :],
                              idx_d.at[pl.ds(e, 1), :], esem).wait()

    def _adjust(e):
        def _adj(g, carry):
            gs = g * LANES
            idx_s[e, pl.ds(gs, LANES)] = idx_s[e, pl.ds(gs, LANES)] + sbase
            idx_d[e, pl.ds(gs, LANES)] = idx_d[e, pl.ds(gs, LANES)] - dbase
            return carry

        lax.fori_loop(0, CHUNK // LANES, _adj, 0)

    def _gather_fire(e, rb):
        return pltpu.async_copy(tin.at[idx_s.at[e]], rows[rb], gsem[rb])

    def _scale(e, rb):
        def _sc(g, carry):
            v = valv[e, pl.ds(g * LANES, LANES)]
            base_row = g * LANES
            for jj in range(LANES):
                rr = base_row + jj
                vb = jnp.full((LANES,), v[jj], jnp.float32)
                rows[rb][rr, pl.ds(0, LANES)] = (
                    rows[rb][rr, pl.ds(0, LANES)] * vb)
                rows[rb][rr, pl.ds(LANES, LANES)] = (
                    rows[rb][rr, pl.ds(LANES, LANES)] * vb)
            return carry

        lax.fori_loop(0, CHUNK // LANES, _sc, 0)

    def _scatter_fire(e, rb):
        return pltpu.async_copy(rows[rb], acc.at[idx_d.at[e]], ssem, add=True)

    for t in range(4):
        _edge_fire(t, t)

    def _body(k, carry):
        i = k * 4
        ib = i & 4
        nb = 4 - ib
        for t in range(4):
            _edge_wait(i + t, ib + t)
        gds = []
        for t in range(4):
            _adjust(ib + t)
            gds.append(_gather_fire(ib + t, t))
        for t in range(4):
            @pl.when(i + 4 + t < N_CHUNKS)
            def _(t=t):
                _edge_fire(i + 4 + t, nb + t)
        sds = []
        for t in range(4):
            gds[t].wait()
            _scale(ib + t, t)
            sds.append(_scatter_fire(ib + t, t))
        for d in sds:
            d.wait()
        return carry

    lax.fori_loop(0, N_CHUNKS // 4, _body, 0)
    _edge_wait(N_CHUNKS - 1, 0)
    _adjust(0)
    gd = _gather_fire(0, 0)
    gd.wait()
    _scale(0, 0)
    _scatter_fire(0, 0).wait()
    plsc.subcore_barrier()

    obase = jnp.where(c == 0, HALF, 0) + row0
    wds = [pltpu.async_copy(acc.at[pl.ds(row0 + k * WB, WB), :],
                            tout.at[pl.ds(obase + k * WB, WB), :], gsem0)
           for k in range(WB_STEPS)]
    for d in wds:
        d.wait()


B_PER_W = BATCH // (NC * NS)


@functools.partial(
    pl.kernel,
    out_type=(
        jax.ShapeDtypeStruct((BATCH, EMB), jnp.float32),
        jax.ShapeDtypeStruct((BATCH, EMB), jnp.float32),
        jax.ShapeDtypeStruct((BATCH, EMB), jnp.float32),
        jax.ShapeDtypeStruct((BATCH, EMB), jnp.float32),
        jax.ShapeDtypeStruct((BATCH, EMB), jnp.float32),
        jax.ShapeDtypeStruct((BATCH, EMB), jnp.float32),
    ),
    mesh=_mesh,
    scratch_types=[
        pltpu.VMEM((B_PER_W,), jnp.int32),
        pltpu.VMEM((B_PER_W, EMB), jnp.float32),
        pltpu.VMEM((B_PER_W, EMB), jnp.float32),
        pltpu.SemaphoreType.DMA,
    ],
    compiler_params=pltpu.CompilerParams(use_tc_tiling_on_sc=False),
)
def _batch_gather(t0, t1, t2, t3, users, items, upop, ipop, eup, eip,
                  su, si, u0, p0, upr, ppr, idxv, accv, tmpv, sem):
    c = lax.axis_index("c")
    s = lax.axis_index("s")
    wid = c * NS + s
    base = wid * B_PER_W

    def _acc_add(i, carry):
        accv[i, pl.ds(0, LANES)] = accv[i, pl.ds(0, LANES)] + tmpv[i, pl.ds(0, LANES)]
        accv[i, pl.ds(LANES, LANES)] = (
            accv[i, pl.ds(LANES, LANES)] + tmpv[i, pl.ds(LANES, LANES)]
        )
        return carry

    def _sum4(out_sum, out_first):
        pltpu.async_copy(t0.at[idxv], accv, sem).wait()
        if out_first is not None:
            pltpu.sync_copy(accv, out_first.at[pl.ds(base, B_PER_W), :])
        for t in (t1, t2, t3):
            pltpu.async_copy(t.at[idxv], tmpv, sem).wait()
            lax.fori_loop(0, B_PER_W, _acc_add, 0)
        pltpu.sync_copy(accv, out_sum.at[pl.ds(base, B_PER_W), :])

    pltpu.sync_copy(users.at[pl.ds(base, B_PER_W)], idxv)
    _sum4(su, u0)

    pltpu.sync_copy(items.at[pl.ds(base, B_PER_W)], idxv)
    for g in range(B_PER_W // LANES):
        gs = g * LANES
        idxv[pl.ds(gs, LANES)] = idxv[pl.ds(gs, LANES)] + HALF
    _sum4(si, p0)

    pltpu.sync_copy(upop.at[pl.ds(base, B_PER_W)], idxv)
    pltpu.async_copy(eup.at[idxv], tmpv, sem).wait()
    pltpu.sync_copy(tmpv, upr.at[pl.ds(base, B_PER_W), :])
    pltpu.sync_copy(ipop.at[pl.ds(base, B_PER_W)], idxv)
    pltpu.async_copy(eip.at[idxv], tmpv, sem).wait()
    pltpu.sync_copy(tmpv, ppr.at[pl.ds(base, B_PER_W), :])


TC_BLK = 512
TC_STEPS = BATCH // TC_BLK


def _normalize_rows(x):
    n = jnp.sqrt(jnp.sum(x * x, axis=-1, keepdims=True))
    return x / jnp.maximum(n, 1e-12)


def _tc_body(su_b, si_f, si_b, u0_b, p0_b, upr_b, ppr_f, ppr_b, out_ref):
    r = pl.program_id(0)
    ue = _normalize_rows(su_b[...] * 0.25)
    pe_all = _normalize_rows(si_f[...] * 0.25)
    pe_b = _normalize_rows(si_b[...] * 0.25)

    ratings = lax.dot_general(ue, pe_all, (((1,), (1,)), ((), ())),
                              preferred_element_type=jnp.float32)
    den1 = jnp.sum(jnp.exp(ratings * (1.0 / TAU1)), axis=1)
    diag = jnp.sum(ue * pe_b, axis=1)
    margin = jnp.sum(upr_b[...] * ppr_b[...], axis=1)
    phi = 1.0 - 1.0 / (1.0 + jnp.exp(-margin))
    x = jnp.clip(diag, -1.0 + 1e-07, 1.0 - 1e-07)
    adj = x * jnp.cos(phi) - jnp.sqrt(1.0 - x * x) * jnp.sin(phi)
    l1 = jnp.sum(jnp.log(den1) - adj * (1.0 / TAU1))

    un = _normalize_rows(upr_b[...])
    pn_all = _normalize_rows(ppr_f[...])
    pn_b = _normalize_rows(ppr_b[...])
    rat2 = lax.dot_general(un, pn_all, (((1,), (1,)), ((), ())),
                           preferred_element_type=jnp.float32)
    den2 = jnp.sum(jnp.exp(rat2 * (1.0 / TAU2)), axis=1)
    diag2 = jnp.sum(un * pn_b, axis=1)
    l2 = jnp.sum(jnp.log(den2) - diag2 * (1.0 / TAU2))

    su0 = jnp.sum(u0_b[...] ** 2)
    sp0 = jnp.sum(p0_b[...] ** 2)
    sun = jnp.sum(un ** 2)
    spn = jnp.sum(pn_b ** 2)

    lane = lax.broadcasted_iota(jnp.int32, (1, 128), 1)
    parts = (l1, l2, su0, sp0, sun, spn)
    vec = jnp.zeros((1, 128), jnp.float32)
    for k, p in enumerate(parts):
        vec = vec + jnp.where(lane == k, p, 0.0)

    @pl.when(r == 0)
    def _():
        out_ref[...] = vec

    @pl.when(r > 0)
    def _():
        out_ref[...] = out_ref[...] + vec

    @pl.when(r == TC_STEPS - 1)
    def _():
        tot = out_ref[...]

        def pick(k):
            return jnp.sum(jnp.where(lane == k, tot, 0.0))

        s0, s1, s2, s3, s4, s5 = (pick(k) for k in range(6))
        loss1 = (1.0 - W_LAMBDA) / BATCH * s0
        loss2 = W_LAMBDA / BATCH * s1
        reg1 = (0.5 * s2 + BATCH * 0.5 * s3) / BATCH
        reg2 = (0.5 * s4 + BATCH * 0.5 * s5) / BATCH
        outs = (loss1, loss2, DECAY * (reg1 + reg2), DECAY * reg2, DECAY * reg1)
        fv = jnp.zeros((1, 128), jnp.float32)
        for k, p in enumerate(outs):
            fv = fv + jnp.where(lane == k, p, 0.0)
        out_ref[...] = fv


_blk = pl.BlockSpec((TC_BLK, EMB), lambda r: (r, 0))
_full = pl.BlockSpec((BATCH, EMB), lambda r: (0, 0))

_tc_loss = pl.pallas_call(
    _tc_body,
    grid=(TC_STEPS,),
    in_specs=[_blk, _full, _blk, _blk, _blk, _blk, _full, _blk],
    out_specs=pl.BlockSpec((1, 128), lambda r: (0, 0)),
    out_shape=jax.ShapeDtypeStruct((1, 128), jnp.float32),
)


def kernel(users, pos_items, users_pop, pos_items_pop, embed_user, embed_item,
           embed_user_pop, embed_item_pop, edge_src, edge_dst, edge_val):
    pad = jnp.zeros((PAD, EMB), jnp.float32)
    t0 = jnp.concatenate([embed_user, pad, embed_item, pad], axis=0)
    dst2 = edge_dst.reshape(-1, SUB)
    t1 = _layer(edge_src, dst2, edge_val, t0)
    t2 = _layer(edge_src, dst2, edge_val, t1)
    t3 = _layer(edge_src, dst2, edge_val, t2)
    su, si, u0, p0, upr, ppr = _batch_gather(
        t0, t1, t2, t3,
        users.astype(jnp.int32), pos_items.astype(jnp.int32),
        users_pop.astype(jnp.int32), pos_items_pop.astype(jnp.int32),
        embed_user_pop, embed_item_pop)
    out = _tc_loss(su, si, si, u0, p0, upr, ppr, ppr)
    return (out[0, 0], out[0, 1], out[0, 2], out[0, 3], out[0, 4])
